# baseline (speedup 1.0000x reference)
import functools

import jax
import jax.numpy as jnp
from jax import lax
from jax.experimental import pallas as pl
from jax.experimental.pallas import tpu as pltpu
from jax.experimental.pallas import tpu_sc as plsc

def _pick_chunk(per_tile):
    for c in (80, 128, 120, 112, 104, 96, 88, 72, 64, 56, 48, 40, 32, 24,
              16, 8):
        if per_tile % c == 0:
            return c
    raise ValueError(per_tile)


def _gather_sc(hidden, rela_embed, sub, rel):
    E = sub.shape[0]
    d = hidden.shape[1]
    mesh = plsc.VectorSubcoreMesh(core_axis_name="c", subcore_axis_name="s")
    NC, NS = 2, 16
    per_tile = E // (NC * NS)
    CHUNK = _pick_chunk(per_tile)
    n_chunks = per_tile // CHUNK
    assert per_tile % CHUNK == 0 and n_chunks >= 3

    @functools.partial(
        pl.kernel,
        mesh=mesh,
        out_type=jax.ShapeDtypeStruct((E, d), jnp.float32),
        scratch_types=[
            pltpu.VMEM((CHUNK,), jnp.int32),
            pltpu.VMEM((CHUNK,), jnp.int32),
            pltpu.VMEM((CHUNK,), jnp.int32),
            pltpu.VMEM((CHUNK,), jnp.int32),
            pltpu.VMEM((CHUNK, d), jnp.float32),
            pltpu.VMEM((CHUNK, d), jnp.float32),
            pltpu.VMEM((CHUNK, d), jnp.float32),
            pltpu.VMEM((CHUNK, d), jnp.float32),
            pltpu.SemaphoreType.DMA,
            pltpu.SemaphoreType.DMA,
            pltpu.SemaphoreType.DMA,
            pltpu.SemaphoreType.DMA,
            pltpu.SemaphoreType.DMA,
            pltpu.SemaphoreType.DMA,
        ],
    )
    def k(hid_hbm, rel_hbm, sub_hbm, reli_hbm, g_out,
          ia0, ia1, ib0, ib1, ra0, ra1, rb0, rb1,
          si0, si1, sg0, sg1, sw0, sw1):
        ci = lax.axis_index("c")
        sci = lax.axis_index("s")
        base = (sci * NC + ci) * per_tile
        idxa, idxb = (ia0, ia1), (ib0, ib1)
        rowsa, rowsb = (ra0, ra1), (rb0, rb1)
        sem_i, sem_g, sem_w = (si0, si1), (sg0, sg1), (sw0, sw1)

        def start_idx(s, off):
            pltpu.make_async_copy(sub_hbm.at[pl.ds(off, CHUNK)],
                                  idxa[s], sem_i[s]).start()
            pltpu.make_async_copy(reli_hbm.at[pl.ds(off, CHUNK)],
                                  idxb[s], sem_i[s]).start()

        def wait_idx(s):
            pltpu.make_async_copy(sub_hbm.at[pl.ds(0, CHUNK)],
                                  idxa[s], sem_i[s]).wait()
            pltpu.make_async_copy(reli_hbm.at[pl.ds(0, CHUNK)],
                                  idxb[s], sem_i[s]).wait()

        def start_gather(s):
            pltpu.make_async_copy(hid_hbm.at[idxa[s]], rowsa[s],
                                  sem_g[s]).start()
            pltpu.make_async_copy(rel_hbm.at[idxb[s]], rowsb[s],
                                  sem_g[s]).start()

        def wait_gather(s):
            pltpu.make_async_copy(hid_hbm.at[idxa[s]], rowsa[s],
                                  sem_g[s]).wait()
            pltpu.make_async_copy(rel_hbm.at[idxb[s]], rowsb[s],
                                  sem_g[s]).wait()

        def start_wb(s, off):
            pltpu.make_async_copy(rowsa[s], g_out.at[pl.ds(off, CHUNK)],
                                  sem_w[s]).start()

        def wait_wb(s):
            pltpu.make_async_copy(rowsa[s], g_out.at[pl.ds(0, CHUNK)],
                                  sem_w[s]).wait()

        def add_rows(s):
            ra, rb = rowsa[s], rowsb[s]

            def body(r, carry):
                for cc in range(d // 16):
                    sl = pl.ds(cc * 16, 16)
                    ra[r, sl] = ra[r, sl] + rb[r, sl]
                return carry

            lax.fori_loop(0, CHUNK, body, 0)

        start_idx(0, base)
        wait_idx(0)
        start_gather(0)
        start_idx(1, base + CHUNK)

        def chunk_body(c, carry):
            off = base + c * CHUNK
            for s in (0, 1):
                o = 1 - s

                @pl.when((c & 1) == s)
                def _():
                    @pl.when(c + 1 < n_chunks)
                    def _():
                        wait_idx(o)

                        @pl.when(c >= 1)
                        def _():
                            wait_wb(o)

                        start_gather(o)

                    wait_gather(s)
                    add_rows(s)
                    start_wb(s, off)

                    @pl.when(c + 2 < n_chunks)
                    def _():
                        start_idx(s, off + 2 * CHUNK)

            return carry

        lax.fori_loop(0, n_chunks, chunk_body, 0)
        wait_wb(0)
        wait_wb(1)

    return k(hidden, rela_embed, sub, rel)


def _dense_tc(g, h_sub, W1, Ws_attn, w_alpha_w, w_alpha_b2, row0):
    E, d = g.shape
    attn = Ws_attn.shape[1]
    BE = 3200
    grid = E // BE
    blk0 = row0 // BE
    assert E % BE == 0 and row0 % BE == 0

    def body(g_ref, hb_ref, w1_ref, ws_ref, ww_ref, wb_ref,
             msg_ref, alpha_ref):
        x = g_ref[...] + hb_ref[...]
        m = jnp.dot(x, w1_ref[...], preferred_element_type=jnp.float32)
        t = jnp.maximum(
            jnp.dot(m, ws_ref[...], preferred_element_type=jnp.float32), 0.0)
        logit = jnp.dot(t, ww_ref[...], preferred_element_type=jnp.float32)
        a = jax.nn.sigmoid(logit + wb_ref[0, 0])
        msg_ref[...] = a * m
        alpha_ref[...] = a

    return pl.pallas_call(
        body,
        grid=(grid,),
        in_specs=[
            pl.BlockSpec((BE, d), lambda i: (i, 0)),
            pl.BlockSpec((BE, d), lambda i: (i + blk0, 0)),
            pl.BlockSpec((d, d), lambda i: (0, 0)),
            pl.BlockSpec((d, attn), lambda i: (0, 0)),
            pl.BlockSpec((attn, 1), lambda i: (0, 0)),
            pl.BlockSpec((1, 1), lambda i: (0, 0)),
        ],
        out_specs=[
            pl.BlockSpec((BE, d), lambda i: (i, 0)),
            pl.BlockSpec((BE, 1), lambda i: (i, 0)),
        ],
        out_shape=[
            jax.ShapeDtypeStruct((E, d), jnp.float32),
            jax.ShapeDtypeStruct((E, 1), jnp.float32),
        ],
    )(g, h_sub, W1, Ws_attn, w_alpha_w, w_alpha_b2)


def _scatter_sc(msg, obj, zeros_init):
    E, d = msg.shape
    n_pad = zeros_init.shape[0]
    mesh = plsc.VectorSubcoreMesh(core_axis_name="c", subcore_axis_name="s")
    NC, NS = 2, 16
    per_tile = E // (NC * NS)
    CHUNK = _pick_chunk(per_tile)
    n_chunks = per_tile // CHUNK
    rows_per_tile = n_pad // NS
    assert per_tile % CHUNK == 0 and n_pad % NS == 0 and rows_per_tile % 8 == 0

    @functools.partial(
        pl.kernel,
        mesh=mesh,
        out_type=jax.ShapeDtypeStruct((NC, n_pad, d), jnp.float32),
        scratch_types=[
            pltpu.VMEM((CHUNK,), jnp.int32),
            pltpu.VMEM((CHUNK,), jnp.int32),
            pltpu.VMEM((CHUNK, d), jnp.float32),
            pltpu.VMEM((CHUNK, d), jnp.float32),
            pltpu.VMEM_SHARED((n_pad, d), jnp.float32),
            pltpu.SemaphoreType.DMA,
            pltpu.SemaphoreType.DMA,
            pltpu.SemaphoreType.DMA,
            pltpu.SemaphoreType.DMA,
        ],
    )
    def k(msg_hbm, obj_hbm, zero_hbm, out_hbm,
          ix0, ix1, rw0, rw1, acc, sl0, sl1, ss0, ss1):
        ci = lax.axis_index("c")
        si = lax.axis_index("s")
        zbase = si * rows_per_tile
        idx, rows = (ix0, ix1), (rw0, rw1)
        sem_l, sem_s = (sl0, sl1), (ss0, ss1)
        pltpu.sync_copy(zero_hbm.at[pl.ds(zbase, rows_per_tile)],
                        acc.at[pl.ds(zbase, rows_per_tile)])
        plsc.subcore_barrier()

        base = (ci * NS + si) * per_tile

        def start_load(s, off):
            pltpu.make_async_copy(obj_hbm.at[pl.ds(off, CHUNK)],
                                  idx[s], sem_l[s]).start()
            pltpu.make_async_copy(msg_hbm.at[pl.ds(off, CHUNK)],
                                  rows[s], sem_l[s]).start()

        def wait_load(s):
            pltpu.make_async_copy(obj_hbm.at[pl.ds(0, CHUNK)],
                                  idx[s], sem_l[s]).wait()
            pltpu.make_async_copy(msg_hbm.at[pl.ds(0, CHUNK)],
                                  rows[s], sem_l[s]).wait()

        def start_scat(s):
            pltpu.async_copy(rows[s], acc.at[idx[s]], sem_s[s], add=True)

        def wait_scat(s):
            pltpu.make_async_copy(rows[s], acc.at[idx[s]], sem_s[s]).wait()

        start_load(0, base)

        def body(c, carry):
            off = base + c * CHUNK
            for s in (0, 1):
                o = 1 - s

                @pl.when((c & 1) == s)
                def _():
                    wait_load(s)

                    @pl.when(c >= 1)
                    def _():
                        wait_scat(o)

                    @pl.when(c + 1 < n_chunks)
                    def _():
                        start_load(o, off + CHUNK)

                    start_scat(s)

            return carry

        lax.fori_loop(0, n_chunks, body, 0)
        wait_scat((n_chunks - 1) & 1)
        plsc.subcore_barrier()
        pltpu.sync_copy(acc.at[pl.ds(zbase, rows_per_tile)],
                        out_hbm.at[ci, pl.ds(zbase, rows_per_tile)])

    return k(msg, obj, zeros_init)


def _final_tc(partials_a, partials_b, W_h, n):
    d = partials_a.shape[2]
    BN = 2000
    grid = n // BN

    def body(p0_ref, p1_ref, p2_ref, p3_ref, wh_ref, out_ref):
        acc = (p0_ref[0] + p1_ref[0]) + (p2_ref[0] + p3_ref[0])
        out_ref[...] = jnp.dot(acc, wh_ref[...],
                               preferred_element_type=jnp.float32)

    return pl.pallas_call(
        body,
        grid=(grid,),
        in_specs=[
            pl.BlockSpec((1, BN, d), lambda i: (0, i, 0)),
            pl.BlockSpec((1, BN, d), lambda i: (1, i, 0)),
            pl.BlockSpec((1, BN, d), lambda i: (0, i, 0)),
            pl.BlockSpec((1, BN, d), lambda i: (1, i, 0)),
            pl.BlockSpec((d, d), lambda i: (0, 0)),
        ],
        out_specs=pl.BlockSpec((BN, d), lambda i: (i, 0)),
        out_shape=jax.ShapeDtypeStruct((n, d), jnp.float32),
    )(partials_a, partials_a, partials_b, partials_b, W_h)


def kernel(q_sub, hidden, edges, n_node, old_nodes_new_idx,
           entity_pretrain_emb, h_sub, rela_embed, W1, Ws_attn,
           w_alpha_w, w_alpha_b, W_h):
    n = hidden.shape[0]
    sub = edges[:, 4].astype(jnp.int32)
    rel = edges[:, 2].astype(jnp.int32)
    obj = jnp.minimum(edges[:, 5], n_node - 1).astype(jnp.int32)

    n_pad = ((n + 127) // 128) * 128
    zeros = jnp.zeros((n_pad, hidden.shape[1]), jnp.float32)
    wb2 = w_alpha_b.reshape(1, 1)
    E = sub.shape[0]
    split = (E * 3 // 5 // 6400) * 6400
    parts, alphas = [], []
    for lo, hi in ((0, split), (split, E)):
        g = _gather_sc(hidden, rela_embed, sub[lo:hi], rel[lo:hi])
        msg, alpha = _dense_tc(g, h_sub, W1, Ws_attn, w_alpha_w, wb2, lo)
        parts.append(_scatter_sc(msg, obj[lo:hi], zeros))
        alphas.append(alpha)
    hidden_new = _final_tc(parts[0], parts[1], W_h, n)
    return (hidden_new, jnp.concatenate(alphas, axis=0))


# speedup vs baseline: 1.0180x; 1.0180x over previous
import functools

import jax
import jax.numpy as jnp
from jax import lax
from jax.experimental import pallas as pl
from jax.experimental.pallas import tpu as pltpu
from jax.experimental.pallas import tpu_sc as plsc

def _pick_chunk(per_tile):
    for c in range(128, 0, -8):
        if per_tile % c == 0:
            return c
    raise ValueError(per_tile)


def _gather_sc(hidden, rela_embed, sub, rel):
    E = sub.shape[0]
    d = hidden.shape[1]
    mesh = plsc.VectorSubcoreMesh(core_axis_name="c", subcore_axis_name="s")
    NC, NS = 2, 16
    per_tile = E // (NC * NS)
    CHUNK = _pick_chunk(per_tile)
    n_chunks = per_tile // CHUNK
    assert per_tile % CHUNK == 0 and n_chunks >= 3

    @functools.partial(
        pl.kernel,
        mesh=mesh,
        out_type=jax.ShapeDtypeStruct((E, d), jnp.float32),
        scratch_types=[
            pltpu.VMEM((CHUNK,), jnp.int32),
            pltpu.VMEM((CHUNK,), jnp.int32),
            pltpu.VMEM((CHUNK,), jnp.int32),
            pltpu.VMEM((CHUNK,), jnp.int32),
            pltpu.VMEM((CHUNK, d), jnp.float32),
            pltpu.VMEM((CHUNK, d), jnp.float32),
            pltpu.VMEM((CHUNK, d), jnp.float32),
            pltpu.VMEM((CHUNK, d), jnp.float32),
            pltpu.SemaphoreType.DMA,
            pltpu.SemaphoreType.DMA,
            pltpu.SemaphoreType.DMA,
            pltpu.SemaphoreType.DMA,
            pltpu.SemaphoreType.DMA,
            pltpu.SemaphoreType.DMA,
        ],
    )
    def k(hid_hbm, rel_hbm, sub_hbm, reli_hbm, g_out,
          ia0, ia1, ib0, ib1, ra0, ra1, rb0, rb1,
          si0, si1, sg0, sg1, sw0, sw1):
        ci = lax.axis_index("c")
        sci = lax.axis_index("s")
        base = (sci * NC + ci) * per_tile
        idxa, idxb = (ia0, ia1), (ib0, ib1)
        rowsa, rowsb = (ra0, ra1), (rb0, rb1)
        sem_i, sem_g, sem_w = (si0, si1), (sg0, sg1), (sw0, sw1)

        def start_idx(s, off):
            pltpu.make_async_copy(sub_hbm.at[pl.ds(off, CHUNK)],
                                  idxa[s], sem_i[s]).start()
            pltpu.make_async_copy(reli_hbm.at[pl.ds(off, CHUNK)],
                                  idxb[s], sem_i[s]).start()

        def wait_idx(s):
            pltpu.make_async_copy(sub_hbm.at[pl.ds(0, CHUNK)],
                                  idxa[s], sem_i[s]).wait()
            pltpu.make_async_copy(reli_hbm.at[pl.ds(0, CHUNK)],
                                  idxb[s], sem_i[s]).wait()

        def start_gather(s):
            pltpu.make_async_copy(hid_hbm.at[idxa[s]], rowsa[s],
                                  sem_g[s]).start()
            pltpu.make_async_copy(rel_hbm.at[idxb[s]], rowsb[s],
                                  sem_g[s]).start()

        def wait_gather(s):
            pltpu.make_async_copy(hid_hbm.at[idxa[s]], rowsa[s],
                                  sem_g[s]).wait()
            pltpu.make_async_copy(rel_hbm.at[idxb[s]], rowsb[s],
                                  sem_g[s]).wait()

        def start_wb(s, off):
            pltpu.make_async_copy(rowsa[s], g_out.at[pl.ds(off, CHUNK)],
                                  sem_w[s]).start()

        def wait_wb(s):
            pltpu.make_async_copy(rowsa[s], g_out.at[pl.ds(0, CHUNK)],
                                  sem_w[s]).wait()

        def add_rows(s):
            ra, rb = rowsa[s], rowsb[s]

            def body(r, carry):
                for cc in range(d // 16):
                    sl = pl.ds(cc * 16, 16)
                    ra[r, sl] = ra[r, sl] + rb[r, sl]
                return carry

            lax.fori_loop(0, CHUNK, body, 0)

        start_idx(0, base)
        wait_idx(0)
        start_gather(0)
        start_idx(1, base + CHUNK)

        def chunk_body(c, carry):
            off = base + c * CHUNK
            for s in (0, 1):
                o = 1 - s

                @pl.when((c & 1) == s)
                def _():
                    @pl.when(c + 1 < n_chunks)
                    def _():
                        wait_idx(o)

                        @pl.when(c >= 1)
                        def _():
                            wait_wb(o)

                        start_gather(o)

                    wait_gather(s)
                    add_rows(s)
                    start_wb(s, off)

                    @pl.when(c + 2 < n_chunks)
                    def _():
                        start_idx(s, off + 2 * CHUNK)

            return carry

        lax.fori_loop(0, n_chunks, chunk_body, 0)
        wait_wb(0)
        wait_wb(1)

    return k(hidden, rela_embed, sub, rel)


def _dense_tc(g, h_sub, W1, Ws_attn, w_alpha_w, w_alpha_b2, row0):
    E, d = g.shape
    attn = Ws_attn.shape[1]
    BE = 3200
    grid = E // BE
    blk0 = row0 // BE
    assert E % BE == 0 and row0 % BE == 0

    def body(g_ref, hb_ref, w1_ref, ws_ref, ww_ref, wb_ref,
             msg_ref, alpha_ref):
        x = g_ref[...] + hb_ref[...]
        m = jnp.dot(x, w1_ref[...], preferred_element_type=jnp.float32)
        t = jnp.maximum(
            jnp.dot(m, ws_ref[...], preferred_element_type=jnp.float32), 0.0)
        logit = jnp.dot(t, ww_ref[...], preferred_element_type=jnp.float32)
        a = jax.nn.sigmoid(logit + wb_ref[0, 0])
        msg_ref[...] = a * m
        alpha_ref[...] = a

    return pl.pallas_call(
        body,
        grid=(grid,),
        in_specs=[
            pl.BlockSpec((BE, d), lambda i: (i, 0)),
            pl.BlockSpec((BE, d), lambda i: (i + blk0, 0)),
            pl.BlockSpec((d, d), lambda i: (0, 0)),
            pl.BlockSpec((d, attn), lambda i: (0, 0)),
            pl.BlockSpec((attn, 1), lambda i: (0, 0)),
            pl.BlockSpec((1, 1), lambda i: (0, 0)),
        ],
        out_specs=[
            pl.BlockSpec((BE, d), lambda i: (i, 0)),
            pl.BlockSpec((BE, 1), lambda i: (i, 0)),
        ],
        out_shape=[
            jax.ShapeDtypeStruct((E, d), jnp.float32),
            jax.ShapeDtypeStruct((E, 1), jnp.float32),
        ],
    )(g, h_sub, W1, Ws_attn, w_alpha_w, w_alpha_b2)


def _scatter_sc(msg, obj, zeros_init):
    E, d = msg.shape
    n_pad = zeros_init.shape[0]
    mesh = plsc.VectorSubcoreMesh(core_axis_name="c", subcore_axis_name="s")
    NC, NS = 2, 16
    per_tile = E // (NC * NS)
    CHUNK = _pick_chunk(per_tile)
    n_chunks = per_tile // CHUNK
    rows_per_tile = n_pad // NS
    assert per_tile % CHUNK == 0 and n_pad % NS == 0 and rows_per_tile % 8 == 0

    @functools.partial(
        pl.kernel,
        mesh=mesh,
        out_type=jax.ShapeDtypeStruct((NC, n_pad, d), jnp.float32),
        scratch_types=[
            pltpu.VMEM((CHUNK,), jnp.int32),
            pltpu.VMEM((CHUNK,), jnp.int32),
            pltpu.VMEM((CHUNK, d), jnp.float32),
            pltpu.VMEM((CHUNK, d), jnp.float32),
            pltpu.VMEM_SHARED((n_pad, d), jnp.float32),
            pltpu.SemaphoreType.DMA,
            pltpu.SemaphoreType.DMA,
            pltpu.SemaphoreType.DMA,
            pltpu.SemaphoreType.DMA,
        ],
    )
    def k(msg_hbm, obj_hbm, zero_hbm, out_hbm,
          ix0, ix1, rw0, rw1, acc, sl0, sl1, ss0, ss1):
        ci = lax.axis_index("c")
        si = lax.axis_index("s")
        zbase = si * rows_per_tile
        idx, rows = (ix0, ix1), (rw0, rw1)
        sem_l, sem_s = (sl0, sl1), (ss0, ss1)
        pltpu.sync_copy(zero_hbm.at[pl.ds(zbase, rows_per_tile)],
                        acc.at[pl.ds(zbase, rows_per_tile)])
        plsc.subcore_barrier()

        base = (ci * NS + si) * per_tile

        def start_load(s, off):
            pltpu.make_async_copy(obj_hbm.at[pl.ds(off, CHUNK)],
                                  idx[s], sem_l[s]).start()
            pltpu.make_async_copy(msg_hbm.at[pl.ds(off, CHUNK)],
                                  rows[s], sem_l[s]).start()

        def wait_load(s):
            pltpu.make_async_copy(obj_hbm.at[pl.ds(0, CHUNK)],
                                  idx[s], sem_l[s]).wait()
            pltpu.make_async_copy(msg_hbm.at[pl.ds(0, CHUNK)],
                                  rows[s], sem_l[s]).wait()

        def start_scat(s):
            pltpu.async_copy(rows[s], acc.at[idx[s]], sem_s[s], add=True)

        def wait_scat(s):
            pltpu.make_async_copy(rows[s], acc.at[idx[s]], sem_s[s]).wait()

        start_load(0, base)

        def body(c, carry):
            off = base + c * CHUNK
            for s in (0, 1):
                o = 1 - s

                @pl.when((c & 1) == s)
                def _():
                    wait_load(s)

                    @pl.when(c >= 1)
                    def _():
                        wait_scat(o)

                    @pl.when(c + 1 < n_chunks)
                    def _():
                        start_load(o, off + CHUNK)

                    start_scat(s)

            return carry

        lax.fori_loop(0, n_chunks, body, 0)
        wait_scat((n_chunks - 1) & 1)
        plsc.subcore_barrier()
        pltpu.sync_copy(acc.at[pl.ds(zbase, rows_per_tile)],
                        out_hbm.at[ci, pl.ds(zbase, rows_per_tile)])

    return k(msg, obj, zeros_init)


def _final_tc(parts, W_h, n):
    d = parts[0].shape[2]
    BN = 2000
    grid = n // BN
    np_ = len(parts)

    def body(*refs):
        prefs, wh_ref, out_ref = refs[:2 * np_], refs[2 * np_], refs[-1]
        acc = prefs[0][0]
        for r in prefs[1:]:
            acc = acc + r[0]
        out_ref[...] = jnp.dot(acc, wh_ref[...],
                               preferred_element_type=jnp.float32)

    in_specs = []
    args = []
    for pa in parts:
        in_specs.append(pl.BlockSpec((1, BN, d), lambda i: (0, i, 0)))
        in_specs.append(pl.BlockSpec((1, BN, d), lambda i: (1, i, 0)))
        args += [pa, pa]
    in_specs.append(pl.BlockSpec((d, d), lambda i: (0, 0)))

    return pl.pallas_call(
        body,
        grid=(grid,),
        in_specs=in_specs,
        out_specs=pl.BlockSpec((BN, d), lambda i: (i, 0)),
        out_shape=jax.ShapeDtypeStruct((n, d), jnp.float32),
    )(*args, W_h)


def kernel(q_sub, hidden, edges, n_node, old_nodes_new_idx,
           entity_pretrain_emb, h_sub, rela_embed, W1, Ws_attn,
           w_alpha_w, w_alpha_b, W_h):
    n = hidden.shape[0]
    sub = edges[:, 4].astype(jnp.int32)
    rel = edges[:, 2].astype(jnp.int32)
    obj = jnp.minimum(edges[:, 5], n_node - 1).astype(jnp.int32)

    n_pad = ((n + 127) // 128) * 128
    zeros = jnp.zeros((n_pad, hidden.shape[1]), jnp.float32)
    wb2 = w_alpha_b.reshape(1, 1)
    E = sub.shape[0]
    s1 = E * 32 // 100 // 12800 * 12800
    s2 = E - s1
    parts, alphas = [], []
    for lo, hi in ((0, s1), (s1, s2), (s2, E)):
        g = _gather_sc(hidden, rela_embed, sub[lo:hi], rel[lo:hi])
        msg, alpha = _dense_tc(g, h_sub, W1, Ws_attn, w_alpha_w, wb2, lo)
        parts.append(_scatter_sc(msg, obj[lo:hi], zeros))
        alphas.append(alpha)
    hidden_new = _final_tc(parts, W_h, n)
    return (hidden_new, jnp.concatenate(alphas, axis=0))


# speedup vs baseline: 1.0575x; 1.0388x over previous
import functools

import jax
import jax.numpy as jnp
from jax import lax
from jax.experimental import pallas as pl
from jax.experimental.pallas import tpu as pltpu
from jax.experimental.pallas import tpu_sc as plsc

def _pick_chunk(per_tile):
    for c in range(128, 0, -8):
        if per_tile % c == 0:
            return c
    raise ValueError(per_tile)


def _gather_sc(hidden, rela_embed, sub, rel):
    E = sub.shape[0]
    d = hidden.shape[1]
    mesh = plsc.VectorSubcoreMesh(core_axis_name="c", subcore_axis_name="s")
    NC, NS = 2, 16
    per_tile = E // (NC * NS)
    CHUNK = _pick_chunk(per_tile)
    n_chunks = per_tile // CHUNK
    assert per_tile % CHUNK == 0 and n_chunks >= 3

    @functools.partial(
        pl.kernel,
        mesh=mesh,
        out_type=jax.ShapeDtypeStruct((E, d), jnp.float32),
        scratch_types=[
            pltpu.VMEM((CHUNK,), jnp.int32),
            pltpu.VMEM((CHUNK,), jnp.int32),
            pltpu.VMEM((CHUNK,), jnp.int32),
            pltpu.VMEM((CHUNK,), jnp.int32),
            pltpu.VMEM((CHUNK, d), jnp.float32),
            pltpu.VMEM((CHUNK, d), jnp.float32),
            pltpu.VMEM((CHUNK, d), jnp.float32),
            pltpu.VMEM((CHUNK, d), jnp.float32),
            pltpu.SemaphoreType.DMA,
            pltpu.SemaphoreType.DMA,
            pltpu.SemaphoreType.DMA,
            pltpu.SemaphoreType.DMA,
            pltpu.SemaphoreType.DMA,
            pltpu.SemaphoreType.DMA,
        ],
    )
    def k(hid_hbm, rel_hbm, sub_hbm, reli_hbm, g_out,
          ia0, ia1, ib0, ib1, ra0, ra1, rb0, rb1,
          si0, si1, sg0, sg1, sw0, sw1):
        ci = lax.axis_index("c")
        sci = lax.axis_index("s")
        base = (sci * NC + ci) * per_tile
        idxa, idxb = (ia0, ia1), (ib0, ib1)
        rowsa, rowsb = (ra0, ra1), (rb0, rb1)
        sem_i, sem_g, sem_w = (si0, si1), (sg0, sg1), (sw0, sw1)

        def start_idx(s, off):
            pltpu.make_async_copy(sub_hbm.at[pl.ds(off, CHUNK)],
                                  idxa[s], sem_i[s]).start()
            pltpu.make_async_copy(reli_hbm.at[pl.ds(off, CHUNK)],
                                  idxb[s], sem_i[s]).start()

        def wait_idx(s):
            pltpu.make_async_copy(sub_hbm.at[pl.ds(0, CHUNK)],
                                  idxa[s], sem_i[s]).wait()
            pltpu.make_async_copy(reli_hbm.at[pl.ds(0, CHUNK)],
                                  idxb[s], sem_i[s]).wait()

        def start_gather(s):
            pltpu.make_async_copy(hid_hbm.at[idxa[s]], rowsa[s],
                                  sem_g[s]).start()
            pltpu.make_async_copy(rel_hbm.at[idxb[s]], rowsb[s],
                                  sem_g[s]).start()

        def wait_gather(s):
            pltpu.make_async_copy(hid_hbm.at[idxa[s]], rowsa[s],
                                  sem_g[s]).wait()
            pltpu.make_async_copy(rel_hbm.at[idxb[s]], rowsb[s],
                                  sem_g[s]).wait()

        def start_wb(s, off):
            pltpu.make_async_copy(rowsa[s], g_out.at[pl.ds(off, CHUNK)],
                                  sem_w[s]).start()

        def wait_wb(s):
            pltpu.make_async_copy(rowsa[s], g_out.at[pl.ds(0, CHUNK)],
                                  sem_w[s]).wait()

        def add_rows(s):
            ra, rb = rowsa[s], rowsb[s]

            def body(r, carry):
                for cc in range(d // 16):
                    sl = pl.ds(cc * 16, 16)
                    ra[r, sl] = ra[r, sl] + rb[r, sl]
                return carry

            lax.fori_loop(0, CHUNK, body, 0)

        start_idx(0, base)
        wait_idx(0)
        start_gather(0)
        start_idx(1, base + CHUNK)

        def chunk_body(c, carry):
            off = base + c * CHUNK
            for s in (0, 1):
                o = 1 - s

                @pl.when((c & 1) == s)
                def _():
                    @pl.when(c + 1 < n_chunks)
                    def _():
                        wait_idx(o)

                        @pl.when(c >= 1)
                        def _():
                            wait_wb(o)

                        start_gather(o)

                    wait_gather(s)
                    add_rows(s)
                    start_wb(s, off)

                    @pl.when(c + 2 < n_chunks)
                    def _():
                        start_idx(s, off + 2 * CHUNK)

            return carry

        lax.fori_loop(0, n_chunks, chunk_body, 0)
        wait_wb(0)
        wait_wb(1)

    return k(hidden, rela_embed, sub, rel)


def _dense_tc(g, h_sub, W1, Ws_attn, w_alpha_w, w_alpha_b2, row0):
    E, d = g.shape
    attn = Ws_attn.shape[1]
    BE = 3200
    grid = E // BE
    blk0 = row0 // BE
    assert E % BE == 0 and row0 % BE == 0

    def body(g_ref, hb_ref, w1_ref, ws_ref, ww_ref, wb_ref,
             msg_ref, alpha_ref):
        x = g_ref[...] + hb_ref[...]
        m = jnp.dot(x, w1_ref[...], preferred_element_type=jnp.float32)
        t = jnp.maximum(
            jnp.dot(m, ws_ref[...], preferred_element_type=jnp.float32), 0.0)
        logit = jnp.dot(t, ww_ref[...], preferred_element_type=jnp.float32)
        a = jax.nn.sigmoid(logit + wb_ref[0, 0])
        msg_ref[...] = a * m
        alpha_ref[...] = a

    return pl.pallas_call(
        body,
        grid=(grid,),
        in_specs=[
            pl.BlockSpec((BE, d), lambda i: (i, 0)),
            pl.BlockSpec((BE, d), lambda i: (i + blk0, 0)),
            pl.BlockSpec((d, d), lambda i: (0, 0)),
            pl.BlockSpec((d, attn), lambda i: (0, 0)),
            pl.BlockSpec((attn, 1), lambda i: (0, 0)),
            pl.BlockSpec((1, 1), lambda i: (0, 0)),
        ],
        out_specs=[
            pl.BlockSpec((BE, d), lambda i: (i, 0)),
            pl.BlockSpec((BE, 1), lambda i: (i, 0)),
        ],
        out_shape=[
            jax.ShapeDtypeStruct((E, d), jnp.float32),
            jax.ShapeDtypeStruct((E, 1), jnp.float32),
        ],
    )(g, h_sub, W1, Ws_attn, w_alpha_w, w_alpha_b2)


def _scatter_sc(msg, obj, zeros_init):
    E, d = msg.shape
    n_pad = zeros_init.shape[0]
    mesh = plsc.VectorSubcoreMesh(core_axis_name="c", subcore_axis_name="s")
    NC, NS = 2, 16
    per_tile = E // (NC * NS)
    CHUNK = _pick_chunk(per_tile)
    n_chunks = per_tile // CHUNK
    rows_per_tile = n_pad // NS
    assert per_tile % CHUNK == 0 and n_pad % NS == 0 and rows_per_tile % 8 == 0

    @functools.partial(
        pl.kernel,
        mesh=mesh,
        out_type=jax.ShapeDtypeStruct((NC, n_pad, d), jnp.float32),
        scratch_types=[
            pltpu.VMEM((CHUNK,), jnp.int32),
            pltpu.VMEM((CHUNK,), jnp.int32),
            pltpu.VMEM((CHUNK, d), jnp.float32),
            pltpu.VMEM((CHUNK, d), jnp.float32),
            pltpu.VMEM_SHARED((n_pad, d), jnp.float32),
            pltpu.SemaphoreType.DMA,
            pltpu.SemaphoreType.DMA,
            pltpu.SemaphoreType.DMA,
            pltpu.SemaphoreType.DMA,
        ],
    )
    def k(msg_hbm, obj_hbm, zero_hbm, out_hbm,
          ix0, ix1, rw0, rw1, acc, sl0, sl1, ss0, ss1):
        ci = lax.axis_index("c")
        si = lax.axis_index("s")
        zbase = si * rows_per_tile
        idx, rows = (ix0, ix1), (rw0, rw1)
        sem_l, sem_s = (sl0, sl1), (ss0, ss1)
        pltpu.sync_copy(zero_hbm.at[pl.ds(zbase, rows_per_tile)],
                        acc.at[pl.ds(zbase, rows_per_tile)])
        plsc.subcore_barrier()

        base = (ci * NS + si) * per_tile

        def start_load(s, off):
            pltpu.make_async_copy(obj_hbm.at[pl.ds(off, CHUNK)],
                                  idx[s], sem_l[s]).start()
            pltpu.make_async_copy(msg_hbm.at[pl.ds(off, CHUNK)],
                                  rows[s], sem_l[s]).start()

        def wait_load(s):
            pltpu.make_async_copy(obj_hbm.at[pl.ds(0, CHUNK)],
                                  idx[s], sem_l[s]).wait()
            pltpu.make_async_copy(msg_hbm.at[pl.ds(0, CHUNK)],
                                  rows[s], sem_l[s]).wait()

        def start_scat(s):
            pltpu.async_copy(rows[s], acc.at[idx[s]], sem_s[s], add=True)

        def wait_scat(s):
            pltpu.make_async_copy(rows[s], acc.at[idx[s]], sem_s[s]).wait()

        start_load(0, base)

        def body(c, carry):
            off = base + c * CHUNK
            for s in (0, 1):
                o = 1 - s

                @pl.when((c & 1) == s)
                def _():
                    wait_load(s)

                    @pl.when(c >= 1)
                    def _():
                        wait_scat(o)

                    @pl.when(c + 1 < n_chunks)
                    def _():
                        start_load(o, off + CHUNK)

                    start_scat(s)

            return carry

        lax.fori_loop(0, n_chunks, body, 0)
        wait_scat((n_chunks - 1) & 1)
        plsc.subcore_barrier()
        pltpu.sync_copy(acc.at[pl.ds(zbase, rows_per_tile)],
                        out_hbm.at[ci, pl.ds(zbase, rows_per_tile)])

    return k(msg, obj, zeros_init)


def _final_tc(parts, W_h, n):
    d = parts[0].shape[2]
    BN = 2000
    grid = n // BN
    np_ = len(parts)

    def body(*refs):
        prefs, wh_ref, out_ref = refs[:2 * np_], refs[2 * np_], refs[-1]
        acc = prefs[0][0]
        for r in prefs[1:]:
            acc = acc + r[0]
        out_ref[...] = jnp.dot(acc, wh_ref[...],
                               preferred_element_type=jnp.float32)

    in_specs = []
    args = []
    for pa in parts:
        in_specs.append(pl.BlockSpec((1, BN, d), lambda i: (0, i, 0)))
        in_specs.append(pl.BlockSpec((1, BN, d), lambda i: (1, i, 0)))
        args += [pa, pa]
    in_specs.append(pl.BlockSpec((d, d), lambda i: (0, 0)))

    return pl.pallas_call(
        body,
        grid=(grid,),
        in_specs=in_specs,
        out_specs=pl.BlockSpec((BN, d), lambda i: (i, 0)),
        out_shape=jax.ShapeDtypeStruct((n, d), jnp.float32),
    )(*args, W_h)


def kernel(q_sub, hidden, edges, n_node, old_nodes_new_idx,
           entity_pretrain_emb, h_sub, rela_embed, W1, Ws_attn,
           w_alpha_w, w_alpha_b, W_h):
    n = hidden.shape[0]
    sub = edges[:, 4].astype(jnp.int32)
    rel = edges[:, 2].astype(jnp.int32)
    obj = jnp.minimum(edges[:, 5], n_node - 1).astype(jnp.int32)

    n_pad = ((n + 127) // 128) * 128
    zeros = jnp.zeros((n_pad, hidden.shape[1]), jnp.float32)
    wb2 = w_alpha_b.reshape(1, 1)
    E = sub.shape[0]
    split = E * 64 // 100 // 6400 * 6400
    parts, alphas = [], []
    for lo, hi in ((0, split), (split, E)):
        g = _gather_sc(hidden, rela_embed, sub[lo:hi], rel[lo:hi])
        msg, alpha = _dense_tc(g, h_sub, W1, Ws_attn, w_alpha_w, wb2, lo)
        parts.append(_scatter_sc(msg, obj[lo:hi], zeros))
        alphas.append(alpha)
    hidden_new = _final_tc(parts, W_h, n)
    return (hidden_new, jnp.concatenate(alphas, axis=0))
